# baseline (speedup 1.0000x reference)
import jax
import jax.numpy as jnp
from jax import lax
from jax.experimental import pallas as pl
from jax.experimental.pallas import tpu as pltpu
from jax.experimental.pallas import tpu_sc as plsc

E = 16
HID = 64
WIDE = E * HID
OUT_G = 8
BLK = 1024

L = 16
NC = 2
NS = 16
NW = NC * NS


def _route_sc(n):
    rw = n // NW
    ng = rw // L
    mesh = plsc.VectorSubcoreMesh(core_axis_name="c", subcore_axis_name="s")

    def body(ox_hbm, oy_hbm, oz_hbm, cb_hbm, cl_hbm,
             ox_v, oy_v, oz_v, cb_v, cl_v):
        wid = lax.axis_index("s") * NC + lax.axis_index("c")
        base = wid * rw
        pltpu.sync_copy(ox_hbm.at[pl.ds(base, rw)], ox_v)
        pltpu.sync_copy(oy_hbm.at[pl.ds(base, rw)], oy_v)
        pltpu.sync_copy(oz_hbm.at[pl.ds(base, rw)], oz_v)
        pltpu.sync_copy(cb_hbm, cb_v)

        cb = [[cb_v[k, e] for e in range(E)] for k in range(3)]

        def group(g, carry):
            vx = ox_v[pl.ds(g * L, L)]
            vy = oy_v[pl.ds(g * L, L)]
            vz = oz_v[pl.ds(g * L, L)]
            dx = vx - cb[0][0]
            dy = vy - cb[1][0]
            dz = vz - cb[2][0]
            best = dx * dx + dy * dy + dz * dz
            besti = jnp.zeros((L,), jnp.int32)
            for e in range(1, E):
                dx = vx - cb[0][e]
                dy = vy - cb[1][e]
                dz = vz - cb[2][e]
                d2 = dx * dx + dy * dy + dz * dz
                upd = d2 < best
                besti = jnp.where(upd, jnp.full((L,), e, jnp.int32), besti)
                best = jnp.where(upd, d2, best)
            cl_v[pl.ds(g * L, L)] = besti
            return carry

        lax.fori_loop(0, ng, group, 0)
        pltpu.sync_copy(cl_v, cl_hbm.at[pl.ds(base, rw)])

    return pl.kernel(
        body,
        mesh=mesh,
        out_type=jax.ShapeDtypeStruct((n,), jnp.int32),
        scratch_types=[
            pltpu.VMEM((rw,), jnp.float32),
            pltpu.VMEM((rw,), jnp.float32),
            pltpu.VMEM((rw,), jnp.float32),
            pltpu.VMEM((3, E, L), jnp.float32),
            pltpu.VMEM((rw,), jnp.int32),
        ],
    )


def _moe_body(cl_ref, d_ref, a_ref, w1d_ref, w1a_ref, b1_ref,
              w2s_ref, b2p_ref, w3c_ref, b3c_ref, out_ref):
    d = d_ref[...]
    a = a_ref[...]
    cluster = cl_ref[...]

    lane128 = jax.lax.broadcasted_iota(jnp.int32, (BLK, 128), 1)
    onehot = (lane128 == cluster).astype(jnp.float32)
    eidx = jax.lax.broadcasted_iota(jnp.int32, (BLK, WIDE), 1) // HID
    mask = (eidx == cluster).astype(jnp.float32)

    h1 = jnp.maximum(
        jnp.dot(d, w1d_ref[...], preferred_element_type=jnp.float32)
        + jnp.dot(a, w1a_ref[...], preferred_element_type=jnp.float32)
        + b1_ref[...], 0.0)
    h2 = jnp.maximum(
        jnp.dot(h1 * mask, w2s_ref[...], preferred_element_type=jnp.float32)
        + jnp.dot(onehot, b2p_ref[...], preferred_element_type=jnp.float32),
        0.0)
    r = jnp.dot(h2, w3c_ref[...], preferred_element_type=jnp.float32) \
        + b3c_ref[...]
    mask8 = (lane128 // OUT_G == cluster).astype(jnp.float32)
    rm = r * mask8
    rm = rm[:, :64] + rm[:, 64:]
    rm = rm[:, :32] + rm[:, 32:]
    rm = rm[:, :16] + rm[:, 16:]
    rm = rm[:, :8] + rm[:, 8:]
    out_ref[...] = jax.nn.sigmoid(rm[:, :3])


def kernel(origins, directions, appearance_embedding, centroids,
           W1, b1, W2, b2, W3, b3):
    n = origins.shape[0]
    o = origins[:, 0, :]
    d = directions[:, 0, :]
    a = appearance_embedding[:, 0, :]

    cbcast = jnp.broadcast_to(centroids.T[:, :, None], (3, E, L))
    cluster = _route_sc(n)(o[:, 0], o[:, 1], o[:, 2], cbcast)
    cl2 = cluster.reshape(n, 1)

    w1cat = W1.transpose(1, 0, 2).reshape(W1.shape[1], WIDE)
    w1d = w1cat[:3]
    w1a = w1cat[3:]
    b1cat = b1.reshape(1, WIDE)

    w2s = W2.reshape(WIDE, HID)
    b2p = jnp.pad(b2, ((0, 128 - E), (0, 0)))
    w3c = jnp.pad(W3, ((0, 0), (0, 0), (0, OUT_G - 3))) \
        .transpose(1, 0, 2).reshape(HID, E * OUT_G)
    b3c = jnp.pad(b3, ((0, 0), (0, OUT_G - 3))).reshape(1, E * OUT_G)

    grid = (n // BLK,)
    out = pl.pallas_call(
        _moe_body,
        grid=grid,
        in_specs=[
            pl.BlockSpec((BLK, 1), lambda i: (i, 0)),
            pl.BlockSpec((BLK, 3), lambda i: (i, 0)),
            pl.BlockSpec((BLK, 32), lambda i: (i, 0)),
            pl.BlockSpec((3, WIDE), lambda i: (0, 0)),
            pl.BlockSpec((32, WIDE), lambda i: (0, 0)),
            pl.BlockSpec((1, WIDE), lambda i: (0, 0)),
            pl.BlockSpec((WIDE, HID), lambda i: (0, 0)),
            pl.BlockSpec((128, HID), lambda i: (0, 0)),
            pl.BlockSpec((HID, 128), lambda i: (0, 0)),
            pl.BlockSpec((1, 128), lambda i: (0, 0)),
        ],
        out_specs=pl.BlockSpec((BLK, 3), lambda i: (i, 0)),
        out_shape=jax.ShapeDtypeStruct((n, 3), jnp.float32),
    )(cl2, d, a, w1d, w1a, b1cat, w2s, b2p, w3c, b3c)
    return out


# speedup vs baseline: 1.1241x; 1.1241x over previous
import jax
import jax.numpy as jnp
from jax import lax
from jax.experimental import pallas as pl
from jax.experimental.pallas import tpu as pltpu
from jax.experimental.pallas import tpu_sc as plsc

E = 16
HID = 64
WIDE = E * HID
OUT_G = 8
BLK = 1024

L = 16
NC = 2
NS = 16
NW = NC * NS


def _route_sc(n):
    rw = n // NW
    ng = rw // L
    mesh = plsc.VectorSubcoreMesh(core_axis_name="c", subcore_axis_name="s")

    def body(ox_hbm, oy_hbm, oz_hbm, cb_hbm, cl_hbm,
             ox_v, oy_v, oz_v, cb_v, cl_v):
        wid = lax.axis_index("s") * NC + lax.axis_index("c")
        base = wid * rw
        pltpu.sync_copy(ox_hbm.at[pl.ds(base, rw)], ox_v)
        pltpu.sync_copy(oy_hbm.at[pl.ds(base, rw)], oy_v)
        pltpu.sync_copy(oz_hbm.at[pl.ds(base, rw)], oz_v)
        pltpu.sync_copy(cb_hbm, cb_v)

        cb = [[cb_v[k, e] for e in range(E)] for k in range(3)]

        def group(g, carry):
            vx = ox_v[pl.ds(g * L, L)]
            vy = oy_v[pl.ds(g * L, L)]
            vz = oz_v[pl.ds(g * L, L)]
            dx = vx - cb[0][0]
            dy = vy - cb[1][0]
            dz = vz - cb[2][0]
            best = dx * dx + dy * dy + dz * dz
            besti = jnp.zeros((L,), jnp.int32)
            for e in range(1, E):
                dx = vx - cb[0][e]
                dy = vy - cb[1][e]
                dz = vz - cb[2][e]
                d2 = dx * dx + dy * dy + dz * dz
                upd = d2 < best
                besti = jnp.where(upd, jnp.full((L,), e, jnp.int32), besti)
                best = jnp.where(upd, d2, best)
            cl_v[pl.ds(g * L, L)] = besti
            return carry

        lax.fori_loop(0, ng, group, 0)
        pltpu.sync_copy(cl_v, cl_hbm.at[pl.ds(base, rw)])

    return pl.kernel(
        body,
        mesh=mesh,
        out_type=jax.ShapeDtypeStruct((n,), jnp.int32),
        scratch_types=[
            pltpu.VMEM((rw,), jnp.float32),
            pltpu.VMEM((rw,), jnp.float32),
            pltpu.VMEM((rw,), jnp.float32),
            pltpu.VMEM((3, E, L), jnp.float32),
            pltpu.VMEM((rw,), jnp.int32),
        ],
    )


def _moe_body(cl_ref, d_ref, a_ref, w1c_ref, b1_ref,
              w2s_ref, b2p_ref, w3c_ref, b3c_ref, out_ref):
    x = jnp.concatenate([d_ref[...], a_ref[...]], axis=1)
    cluster = cl_ref[...]

    lane128 = jax.lax.broadcasted_iota(jnp.int32, (BLK, 128), 1)
    onehot = (lane128 == cluster).astype(jnp.float32)
    eidx = jax.lax.broadcasted_iota(jnp.int32, (BLK, WIDE), 1) // HID
    mask = (eidx == cluster).astype(jnp.float32)

    h1 = jnp.maximum(
        jnp.dot(x, w1c_ref[...], preferred_element_type=jnp.float32)
        + b1_ref[...], 0.0)
    h2 = jnp.maximum(
        jnp.dot(h1 * mask, w2s_ref[...], preferred_element_type=jnp.float32)
        + jnp.dot(onehot, b2p_ref[...], preferred_element_type=jnp.float32),
        0.0)
    r = jnp.dot(h2, w3c_ref[...], preferred_element_type=jnp.float32) \
        + b3c_ref[...]
    mask8 = (lane128 // OUT_G == cluster).astype(jnp.float32)
    rm = r * mask8
    rm = rm[:, :64] + rm[:, 64:]
    rm = rm[:, :32] + rm[:, 32:]
    rm = rm[:, :16] + rm[:, 16:]
    rm = rm[:, :8] + rm[:, 8:]
    out_ref[...] = jax.nn.sigmoid(rm[:, :3])


def kernel(origins, directions, appearance_embedding, centroids,
           W1, b1, W2, b2, W3, b3):
    n = origins.shape[0]
    o = origins[:, 0, :]
    d = directions[:, 0, :]
    a = appearance_embedding[:, 0, :]

    cbcast = jnp.broadcast_to(centroids.T[:, :, None], (3, E, L))
    cluster = _route_sc(n)(o[:, 0], o[:, 1], o[:, 2], cbcast)
    cl2 = cluster.reshape(n, 1)

    w1cat = W1.transpose(1, 0, 2).reshape(W1.shape[1], WIDE)
    b1cat = b1.reshape(1, WIDE)

    w2s = W2.reshape(WIDE, HID)
    b2p = jnp.pad(b2, ((0, 128 - E), (0, 0)))
    w3c = jnp.pad(W3, ((0, 0), (0, 0), (0, OUT_G - 3))) \
        .transpose(1, 0, 2).reshape(HID, E * OUT_G)
    b3c = jnp.pad(b3, ((0, 0), (0, OUT_G - 3))).reshape(1, E * OUT_G)

    grid = (n // BLK,)
    out = pl.pallas_call(
        _moe_body,
        grid=grid,
        in_specs=[
            pl.BlockSpec((BLK, 1), lambda i: (i, 0)),
            pl.BlockSpec((BLK, 3), lambda i: (i, 0)),
            pl.BlockSpec((BLK, 32), lambda i: (i, 0)),
            pl.BlockSpec((35, WIDE), lambda i: (0, 0)),
            pl.BlockSpec((1, WIDE), lambda i: (0, 0)),
            pl.BlockSpec((WIDE, HID), lambda i: (0, 0)),
            pl.BlockSpec((128, HID), lambda i: (0, 0)),
            pl.BlockSpec((HID, 128), lambda i: (0, 0)),
            pl.BlockSpec((1, 128), lambda i: (0, 0)),
        ],
        out_specs=pl.BlockSpec((BLK, 3), lambda i: (i, 0)),
        out_shape=jax.ShapeDtypeStruct((n, 3), jnp.float32),
    )(cl2, d, a, w1cat, b1cat, w2s, b2p, w3c, b3c)
    return out


# speedup vs baseline: 1.1625x; 1.0342x over previous
import jax
import jax.numpy as jnp
from jax import lax
from jax.experimental import pallas as pl
from jax.experimental.pallas import tpu as pltpu
from jax.experimental.pallas import tpu_sc as plsc

E = 16
HID = 64
WIDE = E * HID
OUT_G = 8
BLK = 2048

L = 16
NC = 2
NS = 16
NW = NC * NS


def _route_sc(n):
    rw = n // NW
    ng = rw // L
    mesh = plsc.VectorSubcoreMesh(core_axis_name="c", subcore_axis_name="s")

    def body(ox_hbm, oy_hbm, oz_hbm, cb_hbm, cl_hbm,
             ox_v, oy_v, oz_v, cb_v, cl_v):
        wid = lax.axis_index("s") * NC + lax.axis_index("c")
        base = wid * rw
        pltpu.sync_copy(ox_hbm.at[pl.ds(base, rw)], ox_v)
        pltpu.sync_copy(oy_hbm.at[pl.ds(base, rw)], oy_v)
        pltpu.sync_copy(oz_hbm.at[pl.ds(base, rw)], oz_v)
        pltpu.sync_copy(cb_hbm, cb_v)

        cb = [[cb_v[k, e] for e in range(E)] for k in range(3)]

        def group(g, carry):
            vx = ox_v[pl.ds(g * L, L)]
            vy = oy_v[pl.ds(g * L, L)]
            vz = oz_v[pl.ds(g * L, L)]
            dx = vx - cb[0][0]
            dy = vy - cb[1][0]
            dz = vz - cb[2][0]
            best = dx * dx + dy * dy + dz * dz
            besti = jnp.zeros((L,), jnp.int32)
            for e in range(1, E):
                dx = vx - cb[0][e]
                dy = vy - cb[1][e]
                dz = vz - cb[2][e]
                d2 = dx * dx + dy * dy + dz * dz
                upd = d2 < best
                besti = jnp.where(upd, jnp.full((L,), e, jnp.int32), besti)
                best = jnp.where(upd, d2, best)
            cl_v[pl.ds(g * L, L)] = besti
            return carry

        lax.fori_loop(0, ng, group, 0)
        pltpu.sync_copy(cl_v, cl_hbm.at[pl.ds(base, rw)])

    return pl.kernel(
        body,
        mesh=mesh,
        out_type=jax.ShapeDtypeStruct((n,), jnp.int32),
        scratch_types=[
            pltpu.VMEM((rw,), jnp.float32),
            pltpu.VMEM((rw,), jnp.float32),
            pltpu.VMEM((rw,), jnp.float32),
            pltpu.VMEM((3, E, L), jnp.float32),
            pltpu.VMEM((rw,), jnp.int32),
        ],
    )


def _moe_body(cl_ref, d_ref, a_ref, w1c_ref, b1_ref,
              w2s_ref, b2p_ref, w3c_ref, b3c_ref, out_ref):
    x = jnp.concatenate([d_ref[...], a_ref[...]], axis=1)
    cluster = cl_ref[...]

    lane128 = jax.lax.broadcasted_iota(jnp.int32, (BLK, 128), 1)
    onehot = (lane128 == cluster).astype(jnp.float32)
    eidx = jax.lax.broadcasted_iota(jnp.int32, (BLK, WIDE), 1) // HID
    mask = (eidx == cluster).astype(jnp.float32)

    h1 = jnp.maximum(
        jnp.dot(x, w1c_ref[...], preferred_element_type=jnp.float32)
        + b1_ref[...], 0.0)
    h2 = jnp.maximum(
        jnp.dot(h1 * mask, w2s_ref[...], preferred_element_type=jnp.float32)
        + jnp.dot(onehot, b2p_ref[...], preferred_element_type=jnp.float32),
        0.0)
    r = jnp.dot(h2, w3c_ref[...], preferred_element_type=jnp.float32) \
        + b3c_ref[...]
    mask8 = (lane128 // OUT_G == cluster).astype(jnp.float32)
    rm = r * mask8
    rm = rm[:, :64] + rm[:, 64:]
    rm = rm[:, :32] + rm[:, 32:]
    rm = rm[:, :16] + rm[:, 16:]
    rm = rm[:, :8] + rm[:, 8:]
    out_ref[...] = jax.nn.sigmoid(rm[:, :3])


def kernel(origins, directions, appearance_embedding, centroids,
           W1, b1, W2, b2, W3, b3):
    n = origins.shape[0]
    o = origins[:, 0, :]
    d = directions[:, 0, :]
    a = appearance_embedding[:, 0, :]

    cbcast = jnp.broadcast_to(centroids.T[:, :, None], (3, E, L))
    cluster = _route_sc(n)(o[:, 0], o[:, 1], o[:, 2], cbcast)
    cl2 = cluster.reshape(n, 1)

    w1cat = W1.transpose(1, 0, 2).reshape(W1.shape[1], WIDE)
    b1cat = b1.reshape(1, WIDE)

    w2s = W2.reshape(WIDE, HID)
    b2p = jnp.pad(b2, ((0, 128 - E), (0, 0)))
    w3c = jnp.pad(W3, ((0, 0), (0, 0), (0, OUT_G - 3))) \
        .transpose(1, 0, 2).reshape(HID, E * OUT_G)
    b3c = jnp.pad(b3, ((0, 0), (0, OUT_G - 3))).reshape(1, E * OUT_G)

    grid = (n // BLK,)
    out = pl.pallas_call(
        _moe_body,
        grid=grid,
        in_specs=[
            pl.BlockSpec((BLK, 1), lambda i: (i, 0)),
            pl.BlockSpec((BLK, 3), lambda i: (i, 0)),
            pl.BlockSpec((BLK, 32), lambda i: (i, 0)),
            pl.BlockSpec((35, WIDE), lambda i: (0, 0)),
            pl.BlockSpec((1, WIDE), lambda i: (0, 0)),
            pl.BlockSpec((WIDE, HID), lambda i: (0, 0)),
            pl.BlockSpec((128, HID), lambda i: (0, 0)),
            pl.BlockSpec((HID, 128), lambda i: (0, 0)),
            pl.BlockSpec((1, 128), lambda i: (0, 0)),
        ],
        out_specs=pl.BlockSpec((BLK, 3), lambda i: (i, 0)),
        out_shape=jax.ShapeDtypeStruct((n, 3), jnp.float32),
    )(cl2, d, a, w1cat, b1cat, w2s, b2p, w3c, b3c)
    return out
